# baseline (speedup 1.0000x reference)
import functools
import math

import jax
import jax.numpy as jnp
from jax import lax
from jax.experimental import pallas as pl
from jax.experimental.pallas import tpu as pltpu
from jax.experimental.pallas import tpu_sc as plsc

_SMOOTH = 0.1
_PAD = 0

_B = 4
_S = 8192
_R = _B * _S
_V = 1000

_NW = 32
_RPW = _R // _NW
_CHUNK = 32
_NCH = _RPW // _CHUNK
_NFULL = _V // 16


def _sc_body(pred_hbm, tgt_hbm, out_hbm, tbuf, buf0, buf1, resbuf,
             sem0, sem1, *, eps, c0):
    nc = 2
    wid = lax.axis_index("s") * nc + lax.axis_index("c")
    base = wid * _RPW
    wpb = _S // _RPW
    b0 = wid // wpb
    s0 = (wid % wpb) * _RPW

    pltpu.sync_copy(tgt_hbm.at[b0, pl.ds(s0, _RPW)], tbuf.at[pl.ds(0, _RPW)])

    pltpu.async_copy(pred_hbm.at[b0, pl.ds(s0, _CHUNK)], buf0, sem0)
    pltpu.async_copy(pred_hbm.at[b0, pl.ds(s0 + _CHUNK, _CHUNK)], buf1, sem1)

    iota16 = lax.iota(jnp.int32, 16)
    tailmask = jnp.where(iota16 < 8, 0.0, 1.0)
    zero16 = jnp.zeros((16,), jnp.float32)
    zeros_i = jnp.zeros((16,), jnp.int32)

    def process_chunk(jj, buf, accs):
        acc_s, acc_p0, acc_pt, acc_n1 = accs

        def row_body(r, acc_s):
            pa = buf[r, pl.ds(0, 16)]
            pb = buf[r, pl.ds(16, 16)]
            pc = buf[r, pl.ds(32, 16)]
            pd = buf[r, pl.ds(48, 16)]
            for k in range(4, _NFULL, 4):
                pa = pa + buf[r, pl.ds(16 * k, 16)]
                pb = pb + buf[r, pl.ds(16 * (k + 1), 16)]
                if k + 2 < _NFULL:
                    pc = pc + buf[r, pl.ds(16 * (k + 2), 16)]
                if k + 3 < _NFULL:
                    pd = pd + buf[r, pl.ds(16 * (k + 3), 16)]
            pd = pd + buf[r, pl.ds(_V - 16, 16)] * tailmask
            part = (pa + pb) + (pc + pd)
            t = tbuf[pl.ds(jj * _CHUNK + r, 16)][0]
            return acc_s + jnp.where(t != _PAD, part, zero16)

        acc_s = lax.fori_loop(0, _CHUNK, row_body, acc_s, unroll=False)

        for g in range(_CHUNK // 16):
            rows16 = iota16 + g * 16
            t16 = tbuf[pl.ds(jj * _CHUNK + g * 16, 16)]
            ptv = plsc.load_gather(buf, [rows16, t16])
            p0v = plsc.load_gather(buf, [rows16, zeros_i])
            m = t16 != _PAD
            acc_pt = acc_pt + jnp.where(m, ptv, zero16)
            acc_p0 = acc_p0 + jnp.where(m, p0v, zero16)
            acc_n1 = acc_n1 + jnp.where(m, 1.0, 0.0)
        return acc_s, acc_p0, acc_pt, acc_n1

    def outer(j2, accs):
        for b in range(2):
            jj = 2 * j2 + b
            buf = buf0 if b == 0 else buf1
            sem = sem0 if b == 0 else sem1
            pltpu.make_async_copy(
                pred_hbm.at[b0, pl.ds(s0 + jj * _CHUNK, _CHUNK)], buf, sem
            ).wait()
            accs = process_chunk(jj, buf, accs)

            @pl.when(jj + 2 < _NCH)
            def _():
                pltpu.async_copy(
                    pred_hbm.at[b0, pl.ds(s0 + (jj + 2) * _CHUNK, _CHUNK)],
                    buf, sem)
        return accs

    init = (jnp.zeros((16,), jnp.float32), jnp.zeros((16,), jnp.float32),
            jnp.zeros((16,), jnp.float32), jnp.zeros((16,), jnp.float32))
    acc_s, acc_p0, acc_pt, acc_n1 = lax.fori_loop(
        0, _NCH // 2, outer, init, unroll=False)

    res = (c0 * acc_n1 - eps * (acc_s - acc_p0)
           + (eps - (1.0 - _SMOOTH)) * acc_pt)
    resbuf[...] = res
    pltpu.sync_copy(resbuf, out_hbm.at[wid])


def kernel(pred, target):
    B, S, V = pred.shape
    eps = _SMOOTH / (V - 2)
    c0 = _SMOOTH * math.log(eps) + (1.0 - _SMOOTH) * math.log(1.0 - _SMOOTH)

    mesh = plsc.VectorSubcoreMesh(core_axis_name="c", subcore_axis_name="s")
    sc_fn = functools.partial(
        pl.kernel,
        mesh=mesh,
        out_type=jax.ShapeDtypeStruct((_NW, 16), jnp.float32),
        scratch_types=[
            pltpu.VMEM((_RPW + 16,), jnp.int32),
            pltpu.VMEM((_CHUNK, _V), jnp.float32),
            pltpu.VMEM((_CHUNK, _V), jnp.float32),
            pltpu.VMEM((16,), jnp.float32),
            pltpu.SemaphoreType.DMA,
            pltpu.SemaphoreType.DMA,
        ],
        compiler_params=pltpu.CompilerParams(
            use_tc_tiling_on_sc=True, needs_layout_passes=False),
    )(functools.partial(_sc_body, eps=eps, c0=c0))
    partials = sc_fn(pred, target)
    return jnp.sum(partials)


# speedup vs baseline: 1.1364x; 1.1364x over previous
import functools
import math

import jax
import jax.numpy as jnp
from jax import lax
from jax.experimental import pallas as pl
from jax.experimental.pallas import tpu as pltpu
from jax.experimental.pallas import tpu_sc as plsc

_SMOOTH = 0.1
_PAD = 0

_B = 4
_S = 8192
_R = _B * _S
_V = 1000

_NW = 32
_RPW = _R // _NW
_CHUNK = 32
_NCH = _RPW // _CHUNK
_NFULL = _V // 16


def _sc_body(pred_hbm, tgt_hbm, out_hbm, tbuf, buf0, buf1, resbuf,
             sem0, sem1, *, eps, c0):
    nc = 2
    wid = lax.axis_index("s") * nc + lax.axis_index("c")
    base = wid * _RPW

    pltpu.sync_copy(tgt_hbm.at[pl.ds(base, _RPW)], tbuf.at[pl.ds(0, _RPW)])

    pltpu.async_copy(pred_hbm.at[pl.ds(base, _CHUNK)], buf0, sem0)
    pltpu.async_copy(pred_hbm.at[pl.ds(base + _CHUNK, _CHUNK)], buf1, sem1)

    iota16 = lax.iota(jnp.int32, 16)
    tailmask = jnp.where(iota16 < 8, 0.0, 1.0)
    zero16 = jnp.zeros((16,), jnp.float32)
    zeros_i = jnp.zeros((16,), jnp.int32)

    def process_chunk(jj, buf, accs):
        acc_s, acc_p0, acc_pt, acc_n1 = accs

        def row_body(r, acc_s):
            pa = buf[r, pl.ds(0, 16)]
            pb = buf[r, pl.ds(16, 16)]
            pc = buf[r, pl.ds(32, 16)]
            pd = buf[r, pl.ds(48, 16)]
            for k in range(4, _NFULL, 4):
                pa = pa + buf[r, pl.ds(16 * k, 16)]
                pb = pb + buf[r, pl.ds(16 * (k + 1), 16)]
                if k + 2 < _NFULL:
                    pc = pc + buf[r, pl.ds(16 * (k + 2), 16)]
                if k + 3 < _NFULL:
                    pd = pd + buf[r, pl.ds(16 * (k + 3), 16)]
            pd = pd + buf[r, pl.ds(_V - 16, 16)] * tailmask
            part = (pa + pb) + (pc + pd)
            t = tbuf[pl.ds(jj * _CHUNK + r, 16)][0]
            return acc_s + jnp.where(t != _PAD, part, zero16)

        acc_s = lax.fori_loop(0, _CHUNK, row_body, acc_s, unroll=False)

        for g in range(_CHUNK // 16):
            rows16 = iota16 + g * 16
            t16 = tbuf[pl.ds(jj * _CHUNK + g * 16, 16)]
            ptv = plsc.load_gather(buf, [rows16, t16])
            p0v = plsc.load_gather(buf, [rows16, zeros_i])
            m = t16 != _PAD
            acc_pt = acc_pt + jnp.where(m, ptv, zero16)
            acc_p0 = acc_p0 + jnp.where(m, p0v, zero16)
            acc_n1 = acc_n1 + jnp.where(m, 1.0, 0.0)
        return acc_s, acc_p0, acc_pt, acc_n1

    def outer(j2, accs):
        for b in range(2):
            jj = 2 * j2 + b
            buf = buf0 if b == 0 else buf1
            sem = sem0 if b == 0 else sem1
            pltpu.make_async_copy(
                pred_hbm.at[pl.ds(base + jj * _CHUNK, _CHUNK)], buf, sem
            ).wait()
            accs = process_chunk(jj, buf, accs)

            @pl.when(jj + 2 < _NCH)
            def _():
                pltpu.async_copy(
                    pred_hbm.at[pl.ds(base + (jj + 2) * _CHUNK, _CHUNK)],
                    buf, sem)
        return accs

    init = (jnp.zeros((16,), jnp.float32), jnp.zeros((16,), jnp.float32),
            jnp.zeros((16,), jnp.float32), jnp.zeros((16,), jnp.float32))
    acc_s, acc_p0, acc_pt, acc_n1 = lax.fori_loop(
        0, _NCH // 2, outer, init, unroll=False)

    res = (c0 * acc_n1 - eps * (acc_s - acc_p0)
           + (eps - (1.0 - _SMOOTH)) * acc_pt)
    resbuf[...] = res
    pltpu.sync_copy(resbuf, out_hbm.at[wid])


def kernel(pred, target):
    B, S, V = pred.shape
    pred2 = pred.reshape(B * S, V)
    tgt = target.reshape(B * S)
    eps = _SMOOTH / (V - 2)
    c0 = _SMOOTH * math.log(eps) + (1.0 - _SMOOTH) * math.log(1.0 - _SMOOTH)

    mesh = plsc.VectorSubcoreMesh(core_axis_name="c", subcore_axis_name="s")
    sc_fn = functools.partial(
        pl.kernel,
        mesh=mesh,
        out_type=jax.ShapeDtypeStruct((_NW, 16), jnp.float32),
        scratch_types=[
            pltpu.VMEM((_RPW + 16,), jnp.int32),
            pltpu.VMEM((_CHUNK, _V), jnp.float32),
            pltpu.VMEM((_CHUNK, _V), jnp.float32),
            pltpu.VMEM((16,), jnp.float32),
            pltpu.SemaphoreType.DMA,
            pltpu.SemaphoreType.DMA,
        ],
        compiler_params=pltpu.CompilerParams(
            use_tc_tiling_on_sc=True, needs_layout_passes=False),
    )(functools.partial(_sc_body, eps=eps, c0=c0))
    partials = sc_fn(pred2, tgt)
    return jnp.sum(partials)


# speedup vs baseline: 1.1415x; 1.0045x over previous
import functools
import math

import jax
import jax.numpy as jnp
from jax import lax
from jax.experimental import pallas as pl
from jax.experimental.pallas import tpu as pltpu
from jax.experimental.pallas import tpu_sc as plsc

_SMOOTH = 0.1
_PAD = 0

_R = 32768
_V = 1000

_NW = 32
_RPW = _R // _NW
_CHUNK = 32
_NCH = _RPW // _CHUNK
_NFULL = _V // 16


def _sc_body(pred_hbm, tgt_hbm, out_hbm, tbuf, buf0, buf1, resbuf,
             sem0, sem1, *, eps, c0):
    nc = 2
    wid = lax.axis_index("s") * nc + lax.axis_index("c")
    base = wid * _RPW

    pltpu.sync_copy(tgt_hbm.at[pl.ds(base, _RPW)], tbuf.at[pl.ds(0, _RPW)])

    pltpu.async_copy(pred_hbm.at[pl.ds(base, _CHUNK)], buf0, sem0)
    pltpu.async_copy(pred_hbm.at[pl.ds(base + _CHUNK, _CHUNK)], buf1, sem1)

    iota16 = lax.iota(jnp.int32, 16)
    tailmask = jnp.where(iota16 < 8, 0.0, 1.0)
    zero16 = jnp.zeros((16,), jnp.float32)
    zeros_i = jnp.zeros((16,), jnp.int32)

    def process_chunk(jj, buf, accs):
        acc_s, acc_p0, acc_pt, acc_n1 = accs

        def row_body(r, acc_s):
            pa = buf[r, pl.ds(0, 16)]
            pb = buf[r, pl.ds(16, 16)]
            pc = buf[r, pl.ds(32, 16)]
            pd = buf[r, pl.ds(48, 16)]
            for k in range(4, _NFULL, 4):
                pa = pa + buf[r, pl.ds(16 * k, 16)]
                pb = pb + buf[r, pl.ds(16 * (k + 1), 16)]
                if k + 2 < _NFULL:
                    pc = pc + buf[r, pl.ds(16 * (k + 2), 16)]
                if k + 3 < _NFULL:
                    pd = pd + buf[r, pl.ds(16 * (k + 3), 16)]
            pd = pd + buf[r, pl.ds(_V - 16, 16)] * tailmask
            part = (pa + pb) + (pc + pd)
            t = tbuf[pl.ds(jj * _CHUNK + r, 16)][0]
            return acc_s + jnp.where(t != _PAD, part, zero16)

        acc_s = lax.fori_loop(0, _CHUNK, row_body, acc_s, unroll=False)

        for g in range(_CHUNK // 16):
            rows16 = iota16 + g * 16
            t16 = tbuf[pl.ds(jj * _CHUNK + g * 16, 16)]
            ptv = plsc.load_gather(buf, [rows16, t16])
            p0v = plsc.load_gather(buf, [rows16, zeros_i])
            m = t16 != _PAD
            acc_pt = acc_pt + jnp.where(m, ptv, zero16)
            acc_p0 = acc_p0 + jnp.where(m, p0v, zero16)
            acc_n1 = acc_n1 + jnp.where(m, 1.0, 0.0)
        return acc_s, acc_p0, acc_pt, acc_n1

    def outer(j2, accs):
        for b in range(2):
            jj = 2 * j2 + b
            buf = buf0 if b == 0 else buf1
            sem = sem0 if b == 0 else sem1
            pltpu.make_async_copy(
                pred_hbm.at[pl.ds(base + jj * _CHUNK, _CHUNK)], buf, sem
            ).wait()
            accs = process_chunk(jj, buf, accs)

            @pl.when(jj + 2 < _NCH)
            def _():
                pltpu.async_copy(
                    pred_hbm.at[pl.ds(base + (jj + 2) * _CHUNK, _CHUNK)],
                    buf, sem)
        return accs

    init = (jnp.zeros((16,), jnp.float32), jnp.zeros((16,), jnp.float32),
            jnp.zeros((16,), jnp.float32), jnp.zeros((16,), jnp.float32))
    acc_s, acc_p0, acc_pt, acc_n1 = lax.fori_loop(
        0, _NCH // 2, outer, init, unroll=False)

    res = (c0 * acc_n1 - eps * (acc_s - acc_p0)
           + (eps - (1.0 - _SMOOTH)) * acc_pt)
    resbuf[...] = res
    pltpu.sync_copy(resbuf, out_hbm.at[wid])


def kernel(pred, target):
    B, S, V = pred.shape
    pred2 = pred.reshape(B * S, V)
    tgt = target.reshape(B * S)
    eps = _SMOOTH / (V - 2)
    c0 = _SMOOTH * math.log(eps) + (1.0 - _SMOOTH) * math.log(1.0 - _SMOOTH)

    mesh = plsc.VectorSubcoreMesh(core_axis_name="c", subcore_axis_name="s")
    sc_fn = functools.partial(
        pl.kernel,
        mesh=mesh,
        out_type=jax.ShapeDtypeStruct((_NW, 16), jnp.float32),
        scratch_types=[
            pltpu.VMEM((_RPW + 16,), jnp.int32),
            pltpu.VMEM((_CHUNK, _V), jnp.float32),
            pltpu.VMEM((_CHUNK, _V), jnp.float32),
            pltpu.VMEM((16,), jnp.float32),
            pltpu.SemaphoreType.DMA,
            pltpu.SemaphoreType.DMA,
        ],
        compiler_params=pltpu.CompilerParams(
            use_tc_tiling_on_sc=True, needs_layout_passes=False),
    )(functools.partial(_sc_body, eps=eps, c0=c0))
    partials = sc_fn(pred2, tgt)
    return jnp.sum(partials)
